# Initial kernel scaffold; baseline (speedup 1.0000x reference)
import jax
import jax.numpy as jnp
from jax.experimental import pallas as pl


def kernel(f_s, f_t, idx, contrast_idx, W_s, b_s, W_t, b_t, memory_v1, memory_v2):
    raise NotImplementedError("write your pallas kernel here")


# speedup vs baseline: 16.1147x; 16.1147x over previous
import functools

import jax
import jax.numpy as jnp
from jax import lax
from jax.experimental import pallas as pl
from jax.experimental.pallas import tpu as pltpu
from jax.experimental.pallas import tpu_sc as plsc

EPS = 1e-07
T = 0.07
MOM = 0.5
N_DATA = 100000
K = 4096
KP1 = K + 1
FEAT = 128
BSZ = 256
KPAD = 4224
NC = 2
NS = 16
NW = NC * NS
ROW_BLK = 2000
W_GATHER = 1024
SUB = 128


def _sc_mesh():
    return plsc.VectorSubcoreMesh(core_axis_name="c", subcore_axis_name="s")


def _sc_gather_rows(mem1, mem2, idx):
    rows_per_w = BSZ // NW

    @functools.partial(
        pl.kernel,
        out_type=(
            jax.ShapeDtypeStruct((BSZ, FEAT), jnp.float32),
            jax.ShapeDtypeStruct((BSZ, FEAT), jnp.float32),
        ),
        mesh=_sc_mesh(),
        scratch_types=[
            pltpu.VMEM((rows_per_w,), jnp.int32),
            pltpu.VMEM((rows_per_w, FEAT), jnp.float32),
            pltpu.VMEM((rows_per_w, FEAT), jnp.float32),
            pltpu.SemaphoreType.DMA,
            pltpu.SemaphoreType.DMA,
        ],
    )
    def k(m1_hbm, m2_hbm, idx_hbm, o1_hbm, o2_hbm, idx_v, r1_v, r2_v, s1, s2):
        wid = lax.axis_index("s") * NC + lax.axis_index("c")
        base = wid * rows_per_w
        pltpu.sync_copy(idx_hbm.at[pl.ds(base, rows_per_w)], idx_v)
        c1 = pltpu.async_copy(m1_hbm.at[idx_v], r1_v, s1)
        c2 = pltpu.async_copy(m2_hbm.at[idx_v], r2_v, s2)
        c1.wait()
        c2.wait()
        pltpu.sync_copy(r1_v, o1_hbm.at[pl.ds(base, rows_per_w)])
        pltpu.sync_copy(r2_v, o2_hbm.at[pl.ds(base, rows_per_w)])

    return k(mem1, mem2, idx)


def _embed_body(fs, ft, ws, bs, wt, bt, ic, ir, r1r, r2r, v1o, v2o, d1o, d2o):
    cdims = (((1,), (1,)), ((), ()))
    h1 = lax.dot_general(fs[...], ws[...], cdims,
                         preferred_element_type=jnp.float32) + bs[...]
    v1 = h1 / jnp.sqrt(jnp.sum(h1 * h1, axis=1, keepdims=True))
    h2 = lax.dot_general(ft[...], wt[...], cdims,
                         preferred_element_type=jnp.float32) + bt[...]
    v2 = h2 / jnp.sqrt(jnp.sum(h2 * h2, axis=1, keepdims=True))
    lp1 = r1r[...] * MOM + v1 * (1.0 - MOM)
    lp1 = lp1 / jnp.sqrt(jnp.sum(lp1 * lp1, axis=1, keepdims=True))
    lp2 = r2r[...] * MOM + v2 * (1.0 - MOM)
    lp2 = lp2 / jnp.sqrt(jnp.sum(lp2 * lp2, axis=1, keepdims=True))
    ji = lax.broadcasted_iota(jnp.int32, (BSZ, BSZ), 0)
    jj = lax.broadcasted_iota(jnp.int32, (BSZ, BSZ), 1)
    dup = jnp.logical_and(ic[...] == ir[...], jj > ji)
    alive = 1.0 - jnp.any(dup, axis=1, keepdims=True).astype(jnp.float32)
    v1o[...] = v1.astype(jnp.bfloat16)
    v2o[...] = v2.astype(jnp.bfloat16)
    d1o[...] = ((lp1 - r1r[...]) * alive).astype(jnp.bfloat16)
    d2o[...] = ((lp2 - r2r[...]) * alive).astype(jnp.bfloat16)


def _tc_embed(f_s, f_t, W_s, b_s2, W_t, b_t2, idx_col, idx_row, r1, r2):
    full = lambda s: pl.BlockSpec(s, lambda: (0,) * len(s))
    return pl.pallas_call(
        _embed_body,
        in_specs=[
            full((BSZ, f_s.shape[1])),
            full((BSZ, f_t.shape[1])),
            full((FEAT, f_s.shape[1])),
            full((1, FEAT)),
            full((FEAT, f_t.shape[1])),
            full((1, FEAT)),
            full((BSZ, 1)),
            full((1, BSZ)),
            full((BSZ, FEAT)),
            full((BSZ, FEAT)),
        ],
        out_specs=[
            full((BSZ, FEAT)),
            full((BSZ, FEAT)),
            full((BSZ, FEAT)),
            full((BSZ, FEAT)),
        ],
        out_shape=[
            jax.ShapeDtypeStruct((BSZ, FEAT), jnp.bfloat16),
            jax.ShapeDtypeStruct((BSZ, FEAT), jnp.bfloat16),
            jax.ShapeDtypeStruct((BSZ, FEAT), jnp.bfloat16),
            jax.ShapeDtypeStruct((BSZ, FEAT), jnp.bfloat16),
        ],
    )(f_s, f_t, W_s, b_s2, W_t, b_t2, idx_col, idx_row, r1, r2)


def _big_body(idxr_ref, m1_ref, m2_ref, v1_ref, v2_ref, d1_ref, d2_ref,
              n1_ref, n2_ref, s_ref):
    i = pl.program_id(0)
    m1 = m1_ref[...]
    m2 = m2_ref[...]
    rid = lax.broadcasted_iota(jnp.int32, (ROW_BLK, BSZ), 0) + i * ROW_BLK
    oh = (rid == idxr_ref[...]).astype(jnp.bfloat16)
    cd_oh = (((1,), (0,)), ((), ()))
    n1_ref[...] = m1 + lax.dot_general(oh, d1_ref[...], cd_oh,
                                       preferred_element_type=jnp.float32)
    n2_ref[...] = m2 + lax.dot_general(oh, d2_ref[...], cd_oh,
                                       preferred_element_type=jnp.float32)
    cd_s = (((1,), (1,)), ((), ()))
    s1 = lax.dot_general(m1.astype(jnp.bfloat16), v2_ref[...], cd_s,
                         preferred_element_type=jnp.float32)
    s2 = lax.dot_general(m2.astype(jnp.bfloat16), v1_ref[...], cd_s,
                         preferred_element_type=jnp.float32)
    a = lax.bitcast_convert_type(s1, jnp.uint32) + jnp.uint32(0x8000)
    b = lax.bitcast_convert_type(s2, jnp.uint32) + jnp.uint32(0x8000)
    word = (a >> 16) | (b & jnp.uint32(0xFFFF0000))
    s_ref[...] = lax.bitcast_convert_type(word, jnp.int32)


def _tc_big(mem1, mem2, v1b, v2b, d1b, d2b, idx_row):
    nblk = N_DATA // ROW_BLK
    mem_spec = pl.BlockSpec((ROW_BLK, FEAT), lambda i: (i, 0))
    bfull = lambda s: pl.BlockSpec(s, lambda i: (0,) * len(s))
    return pl.pallas_call(
        _big_body,
        grid=(nblk,),
        in_specs=[
            bfull((1, BSZ)),
            mem_spec,
            mem_spec,
            bfull((BSZ, FEAT)),
            bfull((BSZ, FEAT)),
            bfull((BSZ, FEAT)),
            bfull((BSZ, FEAT)),
        ],
        out_specs=[
            pl.BlockSpec((ROW_BLK, FEAT), lambda i: (i, 0)),
            pl.BlockSpec((ROW_BLK, FEAT), lambda i: (i, 0)),
            pl.BlockSpec((ROW_BLK, BSZ), lambda i: (i, 0)),
        ],
        out_shape=[
            jax.ShapeDtypeStruct((N_DATA, FEAT), jnp.float32),
            jax.ShapeDtypeStruct((N_DATA, FEAT), jnp.float32),
            jax.ShapeDtypeStruct((N_DATA, BSZ), jnp.int32),
        ],
    )(idx_row, mem1, mem2, v1b, v2b, d1b, d2b)


def _sc_gather_vals(tab_flat, ewords):
    ntot = ewords.shape[0]
    per_w = ntot // NW
    w_win = 1024
    n_win = per_w // w_win
    n_sub = w_win // SUB

    @functools.partial(
        pl.kernel,
        out_type=(
            jax.ShapeDtypeStruct((ntot,), jnp.float32),
            jax.ShapeDtypeStruct((ntot,), jnp.float32),
        ),
        mesh=_sc_mesh(),
        scratch_types=[
            pltpu.VMEM((w_win,), jnp.int32),
            pltpu.VMEM((w_win,), jnp.int32),
            pltpu.VMEM((w_win,), jnp.float32),
            pltpu.VMEM((w_win,), jnp.float32),
            pltpu.SemaphoreType.DMA,
        ],
    )
    def k(tab_hbm, idx_hbm, o1_hbm, o2_hbm,
          idx_v, w_v, g1_v, g2_v, sem):
        wid = lax.axis_index("s") * NC + lax.axis_index("c")
        base = wid * per_w

        @pl.loop(0, n_win)
        def _win(t):
            off = base + t * w_win
            pltpu.sync_copy(idx_hbm.at[pl.ds(off, w_win)], idx_v)
            copies = [
                pltpu.async_copy(tab_hbm.at[idx_v.at[pl.ds(s * SUB, SUB)]],
                                 w_v.at[pl.ds(s * SUB, SUB)], sem)
                for s in range(n_sub)
            ]
            for cp in copies:
                cp.wait()

            @pl.loop(0, w_win // 16)
            def _ext(q):
                word = w_v[pl.ds(q * 16, 16)]
                g1_v[pl.ds(q * 16, 16)] = lax.bitcast_convert_type(
                    word << 16, jnp.float32)
                g2_v[pl.ds(q * 16, 16)] = lax.bitcast_convert_type(
                    word & jnp.int32(-65536), jnp.float32)

            pltpu.sync_copy(g1_v, o1_hbm.at[pl.ds(off, w_win)])
            pltpu.sync_copy(g2_v, o2_hbm.at[pl.ds(off, w_win)])

    return k(tab_flat, ewords)


def _loss_body(g1r, g2r, lo):
    c0 = K * (1.0 / N_DATA)
    c = c0 + EPS
    kcol = lax.broadcasted_iota(jnp.int32, (BSZ, KPAD), 1)
    valid = kcol < KP1
    tot = jnp.float32(0.0)
    for gr in (g2r, g1r):
        e = jnp.exp(gr[...] * (1.0 / T))
        e = jnp.where(valid, e, 0.0)
        z = jnp.sum(e) * (float(N_DATA) / (BSZ * KP1))
        px = e / z
        term = jnp.where(kcol == 0,
                         jnp.log(px / (px + c)),
                         jnp.log(c0 / (px + c)))
        term = jnp.where(valid, term, 0.0)
        tot = tot + (-jnp.sum(term) / BSZ)
    lo[...] = jnp.reshape(tot, (1, 1))


def _tc_loss(g1, g2):
    full = lambda s: pl.BlockSpec(s, lambda: (0,) * len(s))
    return pl.pallas_call(
        _loss_body,
        in_specs=[full((BSZ, KPAD)), full((BSZ, KPAD))],
        out_specs=full((1, 1)),
        out_shape=jax.ShapeDtypeStruct((1, 1), jnp.float32),
    )(g1, g2)


def kernel(f_s, f_t, idx, contrast_idx, W_s, b_s, W_t, b_t,
           memory_v1, memory_v2):
    idx32 = idx.astype(jnp.int32)
    cidx = contrast_idx.astype(jnp.int32)
    cpad = jnp.pad(cidx, ((0, 0), (0, KPAD - KP1)))
    e_word = cpad * BSZ + jnp.arange(BSZ, dtype=jnp.int32)[:, None]
    ewords = e_word.reshape(BSZ * KPAD)

    r1, r2 = _sc_gather_rows(memory_v1, memory_v2, idx32)
    v1b, v2b, d1b, d2b = _tc_embed(
        f_s, f_t, W_s, b_s[None, :], W_t, b_t[None, :],
        idx32[:, None], idx32[None, :], r1, r2)
    new_m1, new_m2, s12 = _tc_big(
        memory_v1, memory_v2, v1b, v2b, d1b, d2b, idx32[None, :])
    tab_flat = s12.reshape(BSZ * N_DATA)
    g1f, g2f = _sc_gather_vals(tab_flat, ewords)
    loss = _tc_loss(g1f.reshape(BSZ, KPAD), g2f.reshape(BSZ, KPAD))
    return loss.reshape(1), new_m1, new_m2


# speedup vs baseline: 19.5085x; 1.2106x over previous
import functools

import jax
import jax.numpy as jnp
from jax import lax
from jax.experimental import pallas as pl
from jax.experimental.pallas import tpu as pltpu
from jax.experimental.pallas import tpu_sc as plsc

EPS = 1e-07
T = 0.07
MOM = 0.5
N_DATA = 100000
K = 4096
KP1 = K + 1
FEAT = 128
BSZ = 256
KPAD = 4224
NC = 2
NS = 16
NW = NC * NS
ROW_BLK = 2000
W_GATHER = 1024
SUB = 128


def _sc_mesh():
    return plsc.VectorSubcoreMesh(core_axis_name="c", subcore_axis_name="s")


def _sc_gather_rows(mem1, mem2, idx):
    rows_per_w = BSZ // NW

    @functools.partial(
        pl.kernel,
        out_type=(
            jax.ShapeDtypeStruct((BSZ, FEAT), jnp.float32),
            jax.ShapeDtypeStruct((BSZ, FEAT), jnp.float32),
        ),
        mesh=_sc_mesh(),
        scratch_types=[
            pltpu.VMEM((rows_per_w,), jnp.int32),
            pltpu.VMEM((rows_per_w, FEAT), jnp.float32),
            pltpu.VMEM((rows_per_w, FEAT), jnp.float32),
            pltpu.SemaphoreType.DMA,
            pltpu.SemaphoreType.DMA,
        ],
    )
    def k(m1_hbm, m2_hbm, idx_hbm, o1_hbm, o2_hbm, idx_v, r1_v, r2_v, s1, s2):
        wid = lax.axis_index("s") * NC + lax.axis_index("c")
        base = wid * rows_per_w
        pltpu.sync_copy(idx_hbm.at[pl.ds(base, rows_per_w)], idx_v)
        c1 = pltpu.async_copy(m1_hbm.at[idx_v], r1_v, s1)
        c2 = pltpu.async_copy(m2_hbm.at[idx_v], r2_v, s2)
        c1.wait()
        c2.wait()
        pltpu.sync_copy(r1_v, o1_hbm.at[pl.ds(base, rows_per_w)])
        pltpu.sync_copy(r2_v, o2_hbm.at[pl.ds(base, rows_per_w)])

    return k(mem1, mem2, idx)


def _embed_body(fs, ft, ws, bs, wt, bt, ic, ir, r1r, r2r, v1o, v2o, d1o, d2o):
    cdims = (((1,), (1,)), ((), ()))
    h1 = lax.dot_general(fs[...], ws[...], cdims,
                         preferred_element_type=jnp.float32) + bs[...]
    v1 = h1 / jnp.sqrt(jnp.sum(h1 * h1, axis=1, keepdims=True))
    h2 = lax.dot_general(ft[...], wt[...], cdims,
                         preferred_element_type=jnp.float32) + bt[...]
    v2 = h2 / jnp.sqrt(jnp.sum(h2 * h2, axis=1, keepdims=True))
    lp1 = r1r[...] * MOM + v1 * (1.0 - MOM)
    lp1 = lp1 / jnp.sqrt(jnp.sum(lp1 * lp1, axis=1, keepdims=True))
    lp2 = r2r[...] * MOM + v2 * (1.0 - MOM)
    lp2 = lp2 / jnp.sqrt(jnp.sum(lp2 * lp2, axis=1, keepdims=True))
    ji = lax.broadcasted_iota(jnp.int32, (BSZ, BSZ), 0)
    jj = lax.broadcasted_iota(jnp.int32, (BSZ, BSZ), 1)
    dup = jnp.logical_and(ic[...] == ir[...], jj > ji)
    alive = 1.0 - jnp.any(dup, axis=1, keepdims=True).astype(jnp.float32)
    v1o[...] = v1.astype(jnp.bfloat16)
    v2o[...] = v2.astype(jnp.bfloat16)
    d1o[...] = ((lp1 - r1r[...]) * alive).astype(jnp.bfloat16)
    d2o[...] = ((lp2 - r2r[...]) * alive).astype(jnp.bfloat16)


def _tc_embed(f_s, f_t, W_s, b_s2, W_t, b_t2, idx_col, idx_row, r1, r2):
    full = lambda s: pl.BlockSpec(s, lambda: (0,) * len(s))
    return pl.pallas_call(
        _embed_body,
        in_specs=[
            full((BSZ, f_s.shape[1])),
            full((BSZ, f_t.shape[1])),
            full((FEAT, f_s.shape[1])),
            full((1, FEAT)),
            full((FEAT, f_t.shape[1])),
            full((1, FEAT)),
            full((BSZ, 1)),
            full((1, BSZ)),
            full((BSZ, FEAT)),
            full((BSZ, FEAT)),
        ],
        out_specs=[
            full((BSZ, FEAT)),
            full((BSZ, FEAT)),
            full((BSZ, FEAT)),
            full((BSZ, FEAT)),
        ],
        out_shape=[
            jax.ShapeDtypeStruct((BSZ, FEAT), jnp.bfloat16),
            jax.ShapeDtypeStruct((BSZ, FEAT), jnp.bfloat16),
            jax.ShapeDtypeStruct((BSZ, FEAT), jnp.bfloat16),
            jax.ShapeDtypeStruct((BSZ, FEAT), jnp.bfloat16),
        ],
    )(f_s, f_t, W_s, b_s2, W_t, b_t2, idx_col, idx_row, r1, r2)


def _pack_words(s1, s2):
    a = lax.bitcast_convert_type(s1, jnp.uint32) + jnp.uint32(0x8000)
    b = lax.bitcast_convert_type(s2, jnp.uint32) + jnp.uint32(0x8000)
    word = (a >> 16) | (b & jnp.uint32(0xFFFF0000))
    return lax.bitcast_convert_type(word, jnp.int32)


def _big_body(idxr_ref, m1_ref, m2_ref, v1_ref, v2_ref, d1_ref, d2_ref,
              n1_ref, n2_ref, sa_ref, sb_ref):
    i = pl.program_id(0)
    m1 = m1_ref[...]
    m2 = m2_ref[...]
    rid = lax.broadcasted_iota(jnp.int32, (ROW_BLK, BSZ), 0) + i * ROW_BLK
    oh = (rid == idxr_ref[...]).astype(jnp.bfloat16)
    cd_oh = (((1,), (0,)), ((), ()))
    n1_ref[...] = m1 + lax.dot_general(oh, d1_ref[...], cd_oh,
                                       preferred_element_type=jnp.float32)
    n2_ref[...] = m2 + lax.dot_general(oh, d2_ref[...], cd_oh,
                                       preferred_element_type=jnp.float32)
    cd_s = (((1,), (1,)), ((), ()))
    m1b = m1.astype(jnp.bfloat16)
    m2b = m2.astype(jnp.bfloat16)
    half = BSZ // 2
    s1a = lax.dot_general(m1b, v2_ref[0:half, :], cd_s,
                          preferred_element_type=jnp.float32)
    s2a = lax.dot_general(m2b, v1_ref[0:half, :], cd_s,
                          preferred_element_type=jnp.float32)
    s1b = lax.dot_general(m1b, v2_ref[half:BSZ, :], cd_s,
                          preferred_element_type=jnp.float32)
    s2b = lax.dot_general(m2b, v1_ref[half:BSZ, :], cd_s,
                          preferred_element_type=jnp.float32)
    sa_ref[...] = _pack_words(s1a, s2a)
    sb_ref[...] = _pack_words(s1b, s2b)


def _tc_big(mem1, mem2, v1b, v2b, d1b, d2b, idx_row):
    nblk = N_DATA // ROW_BLK
    mem_spec = pl.BlockSpec((ROW_BLK, FEAT), lambda i: (i, 0))
    bfull = lambda s: pl.BlockSpec(s, lambda i: (0,) * len(s))
    half = BSZ // 2
    return pl.pallas_call(
        _big_body,
        grid=(nblk,),
        in_specs=[
            bfull((1, BSZ)),
            mem_spec,
            mem_spec,
            bfull((BSZ, FEAT)),
            bfull((BSZ, FEAT)),
            bfull((BSZ, FEAT)),
            bfull((BSZ, FEAT)),
        ],
        out_specs=[
            pl.BlockSpec((ROW_BLK, FEAT), lambda i: (i, 0)),
            pl.BlockSpec((ROW_BLK, FEAT), lambda i: (i, 0)),
            pl.BlockSpec((ROW_BLK, half), lambda i: (i, 0)),
            pl.BlockSpec((ROW_BLK, half), lambda i: (i, 0)),
        ],
        out_shape=[
            jax.ShapeDtypeStruct((N_DATA, FEAT), jnp.float32),
            jax.ShapeDtypeStruct((N_DATA, FEAT), jnp.float32),
            jax.ShapeDtypeStruct((N_DATA, half), jnp.int32),
            jax.ShapeDtypeStruct((N_DATA, half), jnp.int32),
        ],
    )(idx_row, mem1, mem2, v1b, v2b, d1b, d2b)


def _sc_gather_vals(tabA_flat, tabB_flat, ewords):
    ntot = ewords.shape[0]
    per_w = ntot // NW
    w_win = 1024
    n_win = per_w // w_win
    n_sub = w_win // SUB

    @functools.partial(
        pl.kernel,
        out_type=(
            jax.ShapeDtypeStruct((ntot,), jnp.float32),
            jax.ShapeDtypeStruct((ntot,), jnp.float32),
        ),
        mesh=_sc_mesh(),
        scratch_types=[
            pltpu.VMEM((w_win,), jnp.int32),
            pltpu.VMEM((w_win,), jnp.int32),
            pltpu.VMEM((w_win,), jnp.float32),
            pltpu.VMEM((w_win,), jnp.float32),
            pltpu.SemaphoreType.DMA,
        ],
    )
    def k(tabA_hbm, tabB_hbm, idx_hbm, o1_hbm, o2_hbm,
          idx_v, w_v, g1_v, g2_v, sem):
        wid = lax.axis_index("s") * NC + lax.axis_index("c")
        base = wid * per_w

        @pl.loop(0, n_win)
        def _win(t):
            off = base + t * w_win
            pltpu.sync_copy(idx_hbm.at[pl.ds(off, w_win)], idx_v)

            @pl.when(wid < NW // 2)
            def _a():
                copies = [
                    pltpu.async_copy(
                        tabA_hbm.at[idx_v.at[pl.ds(s * SUB, SUB)]],
                        w_v.at[pl.ds(s * SUB, SUB)], sem)
                    for s in range(n_sub)
                ]
                for cp in copies:
                    cp.wait()

            @pl.when(wid >= NW // 2)
            def _b():
                copies = [
                    pltpu.async_copy(
                        tabB_hbm.at[idx_v.at[pl.ds(s * SUB, SUB)]],
                        w_v.at[pl.ds(s * SUB, SUB)], sem)
                    for s in range(n_sub)
                ]
                for cp in copies:
                    cp.wait()

            @pl.loop(0, w_win // 16)
            def _ext(q):
                word = w_v[pl.ds(q * 16, 16)]
                g1_v[pl.ds(q * 16, 16)] = lax.bitcast_convert_type(
                    word << 16, jnp.float32)
                g2_v[pl.ds(q * 16, 16)] = lax.bitcast_convert_type(
                    word & jnp.int32(-65536), jnp.float32)

            pltpu.sync_copy(g1_v, o1_hbm.at[pl.ds(off, w_win)])
            pltpu.sync_copy(g2_v, o2_hbm.at[pl.ds(off, w_win)])

    return k(tabA_flat, tabB_flat, ewords)


def _loss_body(g1r, g2r, lo):
    c0 = K * (1.0 / N_DATA)
    c = c0 + EPS
    kcol = lax.broadcasted_iota(jnp.int32, (BSZ, KPAD), 1)
    valid = kcol < KP1
    tot = jnp.float32(0.0)
    for gr in (g2r, g1r):
        e = jnp.exp(gr[...] * (1.0 / T))
        e = jnp.where(valid, e, 0.0)
        z = jnp.sum(e) * (float(N_DATA) / (BSZ * KP1))
        px = e / z
        term = jnp.where(kcol == 0,
                         jnp.log(px / (px + c)),
                         jnp.log(c0 / (px + c)))
        term = jnp.where(valid, term, 0.0)
        tot = tot + (-jnp.sum(term) / BSZ)
    lo[...] = jnp.reshape(tot, (1, 1))


def _tc_loss(g1, g2):
    full = lambda s: pl.BlockSpec(s, lambda: (0,) * len(s))
    return pl.pallas_call(
        _loss_body,
        in_specs=[full((BSZ, KPAD)), full((BSZ, KPAD))],
        out_specs=full((1, 1)),
        out_shape=jax.ShapeDtypeStruct((1, 1), jnp.float32),
    )(g1, g2)


def kernel(f_s, f_t, idx, contrast_idx, W_s, b_s, W_t, b_t,
           memory_v1, memory_v2):
    idx32 = idx.astype(jnp.int32)
    cidx = contrast_idx.astype(jnp.int32)
    cpad = jnp.pad(cidx, ((0, 0), (0, KPAD - KP1)))
    half = BSZ // 2
    bmod = jnp.arange(BSZ, dtype=jnp.int32)[:, None] % half
    e_word = cpad * half + bmod
    ewords = e_word.reshape(BSZ * KPAD)

    r1, r2 = _sc_gather_rows(memory_v1, memory_v2, idx32)
    v1b, v2b, d1b, d2b = _tc_embed(
        f_s, f_t, W_s, b_s[None, :], W_t, b_t[None, :],
        idx32[:, None], idx32[None, :], r1, r2)
    new_m1, new_m2, sA, sB = _tc_big(
        memory_v1, memory_v2, v1b, v2b, d1b, d2b, idx32[None, :])
    g1f, g2f = _sc_gather_vals(sA.reshape(N_DATA * half),
                               sB.reshape(N_DATA * half), ewords)
    loss = _tc_loss(g1f.reshape(BSZ, KPAD), g2f.reshape(BSZ, KPAD))
    return loss.reshape(1), new_m1, new_m2


# speedup vs baseline: 21.3313x; 1.0934x over previous
import functools

import jax
import jax.numpy as jnp
from jax import lax
from jax.experimental import pallas as pl
from jax.experimental.pallas import tpu as pltpu
from jax.experimental.pallas import tpu_sc as plsc

EPS = 1e-07
T = 0.07
MOM = 0.5
N_DATA = 100000
K = 4096
KP1 = K + 1
FEAT = 128
BSZ = 256
KPAD = 4224
NC = 2
NS = 16
NW = NC * NS
ROW_BLK = 2000
W_GATHER = 1024
SUB = 128


def _sc_mesh():
    return plsc.VectorSubcoreMesh(core_axis_name="c", subcore_axis_name="s")


def _sc_gather_rows(mem1, mem2, idx):
    rows_per_w = BSZ // NW

    @functools.partial(
        pl.kernel,
        out_type=(
            jax.ShapeDtypeStruct((BSZ, FEAT), jnp.float32),
            jax.ShapeDtypeStruct((BSZ, FEAT), jnp.float32),
        ),
        mesh=_sc_mesh(),
        scratch_types=[
            pltpu.VMEM((rows_per_w,), jnp.int32),
            pltpu.VMEM((rows_per_w, FEAT), jnp.float32),
            pltpu.VMEM((rows_per_w, FEAT), jnp.float32),
            pltpu.SemaphoreType.DMA,
            pltpu.SemaphoreType.DMA,
        ],
    )
    def k(m1_hbm, m2_hbm, idx_hbm, o1_hbm, o2_hbm, idx_v, r1_v, r2_v, s1, s2):
        wid = lax.axis_index("s") * NC + lax.axis_index("c")
        base = wid * rows_per_w
        pltpu.sync_copy(idx_hbm.at[pl.ds(base, rows_per_w)], idx_v)
        c1 = pltpu.async_copy(m1_hbm.at[idx_v], r1_v, s1)
        c2 = pltpu.async_copy(m2_hbm.at[idx_v], r2_v, s2)
        c1.wait()
        c2.wait()
        pltpu.sync_copy(r1_v, o1_hbm.at[pl.ds(base, rows_per_w)])
        pltpu.sync_copy(r2_v, o2_hbm.at[pl.ds(base, rows_per_w)])

    return k(mem1, mem2, idx)


def _embed_body(fs, ft, ws, bs, wt, bt, ic, ir, r1r, r2r, v1o, v2o, d1o, d2o):
    cdims = (((1,), (1,)), ((), ()))
    h1 = lax.dot_general(fs[...], ws[...], cdims,
                         preferred_element_type=jnp.float32) + bs[...]
    v1 = h1 / jnp.sqrt(jnp.sum(h1 * h1, axis=1, keepdims=True))
    h2 = lax.dot_general(ft[...], wt[...], cdims,
                         preferred_element_type=jnp.float32) + bt[...]
    v2 = h2 / jnp.sqrt(jnp.sum(h2 * h2, axis=1, keepdims=True))
    lp1 = r1r[...] * MOM + v1 * (1.0 - MOM)
    lp1 = lp1 / jnp.sqrt(jnp.sum(lp1 * lp1, axis=1, keepdims=True))
    lp2 = r2r[...] * MOM + v2 * (1.0 - MOM)
    lp2 = lp2 / jnp.sqrt(jnp.sum(lp2 * lp2, axis=1, keepdims=True))
    ji = lax.broadcasted_iota(jnp.int32, (BSZ, BSZ), 0)
    jj = lax.broadcasted_iota(jnp.int32, (BSZ, BSZ), 1)
    dup = jnp.logical_and(ic[...] == ir[...], jj > ji)
    alive = 1.0 - jnp.any(dup, axis=1, keepdims=True).astype(jnp.float32)
    v1o[...] = v1.astype(jnp.bfloat16)
    v2o[...] = v2.astype(jnp.bfloat16)
    d1o[...] = ((lp1 - r1r[...]) * alive).astype(jnp.bfloat16)
    d2o[...] = ((lp2 - r2r[...]) * alive).astype(jnp.bfloat16)


def _tc_embed(f_s, f_t, W_s, b_s2, W_t, b_t2, idx_col, idx_row, r1, r2):
    full = lambda s: pl.BlockSpec(s, lambda: (0,) * len(s))
    return pl.pallas_call(
        _embed_body,
        in_specs=[
            full((BSZ, f_s.shape[1])),
            full((BSZ, f_t.shape[1])),
            full((FEAT, f_s.shape[1])),
            full((1, FEAT)),
            full((FEAT, f_t.shape[1])),
            full((1, FEAT)),
            full((BSZ, 1)),
            full((1, BSZ)),
            full((BSZ, FEAT)),
            full((BSZ, FEAT)),
        ],
        out_specs=[
            full((BSZ, FEAT)),
            full((BSZ, FEAT)),
            full((BSZ, FEAT)),
            full((BSZ, FEAT)),
        ],
        out_shape=[
            jax.ShapeDtypeStruct((BSZ, FEAT), jnp.bfloat16),
            jax.ShapeDtypeStruct((BSZ, FEAT), jnp.bfloat16),
            jax.ShapeDtypeStruct((BSZ, FEAT), jnp.bfloat16),
            jax.ShapeDtypeStruct((BSZ, FEAT), jnp.bfloat16),
        ],
    )(f_s, f_t, W_s, b_s2, W_t, b_t2, idx_col, idx_row, r1, r2)


def _pack_words(s1, s2):
    a = lax.bitcast_convert_type(s1, jnp.uint32) + jnp.uint32(0x8000)
    b = lax.bitcast_convert_type(s2, jnp.uint32) + jnp.uint32(0x8000)
    word = (a >> 16) | (b & jnp.uint32(0xFFFF0000))
    return lax.bitcast_convert_type(word, jnp.int32)


def _big_body(idxr_ref, m1_ref, m2_ref, v1_ref, v2_ref, d1_ref, d2_ref,
              n1_ref, n2_ref, sa_ref, sb_ref):
    i = pl.program_id(0)
    m1 = m1_ref[...]
    m2 = m2_ref[...]
    rid = lax.broadcasted_iota(jnp.int32, (ROW_BLK, BSZ), 0) + i * ROW_BLK
    oh = (rid == idxr_ref[...]).astype(jnp.bfloat16)
    cd_oh = (((1,), (0,)), ((), ()))
    n1_ref[...] = m1 + lax.dot_general(oh, d1_ref[...], cd_oh,
                                       preferred_element_type=jnp.float32)
    n2_ref[...] = m2 + lax.dot_general(oh, d2_ref[...], cd_oh,
                                       preferred_element_type=jnp.float32)
    cd_s = (((1,), (1,)), ((), ()))
    m1b = m1.astype(jnp.bfloat16)
    m2b = m2.astype(jnp.bfloat16)
    half = BSZ // 2
    s1a = lax.dot_general(m1b, v2_ref[0:half, :], cd_s,
                          preferred_element_type=jnp.float32)
    s2a = lax.dot_general(m2b, v1_ref[0:half, :], cd_s,
                          preferred_element_type=jnp.float32)
    s1b = lax.dot_general(m1b, v2_ref[half:BSZ, :], cd_s,
                          preferred_element_type=jnp.float32)
    s2b = lax.dot_general(m2b, v1_ref[half:BSZ, :], cd_s,
                          preferred_element_type=jnp.float32)
    sa_ref[...] = _pack_words(s1a, s2a)
    sb_ref[...] = _pack_words(s1b, s2b)


def _tc_big(mem1, mem2, v1b, v2b, d1b, d2b, idx_row):
    nblk = N_DATA // ROW_BLK
    mem_spec = pl.BlockSpec((ROW_BLK, FEAT), lambda i: (i, 0))
    bfull = lambda s: pl.BlockSpec(s, lambda i: (0,) * len(s))
    half = BSZ // 2
    return pl.pallas_call(
        _big_body,
        grid=(nblk,),
        in_specs=[
            bfull((1, BSZ)),
            mem_spec,
            mem_spec,
            bfull((BSZ, FEAT)),
            bfull((BSZ, FEAT)),
            bfull((BSZ, FEAT)),
            bfull((BSZ, FEAT)),
        ],
        out_specs=[
            pl.BlockSpec((ROW_BLK, FEAT), lambda i: (i, 0)),
            pl.BlockSpec((ROW_BLK, FEAT), lambda i: (i, 0)),
            pl.BlockSpec((ROW_BLK, half), lambda i: (i, 0)),
            pl.BlockSpec((ROW_BLK, half), lambda i: (i, 0)),
        ],
        out_shape=[
            jax.ShapeDtypeStruct((N_DATA, FEAT), jnp.float32),
            jax.ShapeDtypeStruct((N_DATA, FEAT), jnp.float32),
            jax.ShapeDtypeStruct((N_DATA, half), jnp.int32),
            jax.ShapeDtypeStruct((N_DATA, half), jnp.int32),
        ],
    )(idx_row, mem1, mem2, v1b, v2b, d1b, d2b)


def _sc_gather_vals(tabA_flat, tabB_flat, ewords):
    ntot = ewords.shape[0]
    per_w = ntot // NW
    w_win = 1408
    n_win = per_w // w_win
    n_sub = w_win // SUB
    n2 = n_win // 2

    @functools.partial(
        pl.kernel,
        out_type=(
            jax.ShapeDtypeStruct((ntot,), jnp.float32),
            jax.ShapeDtypeStruct((ntot,), jnp.float32),
        ),
        mesh=_sc_mesh(),
        scratch_types=[
            pltpu.VMEM((w_win,), jnp.int32),
            pltpu.VMEM((w_win,), jnp.int32),
            pltpu.VMEM((w_win,), jnp.int32),
            pltpu.VMEM((w_win,), jnp.int32),
            pltpu.VMEM((w_win,), jnp.float32),
            pltpu.VMEM((w_win,), jnp.float32),
            pltpu.VMEM((w_win,), jnp.float32),
            pltpu.VMEM((w_win,), jnp.float32),
            pltpu.SemaphoreType.DMA,
            pltpu.SemaphoreType.DMA,
            pltpu.SemaphoreType.DMA,
            pltpu.SemaphoreType.DMA,
            pltpu.SemaphoreType.DMA,
            pltpu.SemaphoreType.DMA,
        ],
    )
    def k(tabA_hbm, tabB_hbm, idx_hbm, o1_hbm, o2_hbm,
          idx0, idx1, w0, w1, g1_0, g2_0, g1_1, g2_1,
          si0, si1, sg0, sg1, so0, so1):
        wid = lax.axis_index("s") * NC + lax.axis_index("c")
        base = wid * per_w

        def fire_gathers(idx_v, w_v, sem):
            @pl.when(wid < NW // 2)
            def _a():
                for s in range(n_sub):
                    pltpu.async_copy(
                        tabA_hbm.at[idx_v.at[pl.ds(s * SUB, SUB)]],
                        w_v.at[pl.ds(s * SUB, SUB)], sem)

            @pl.when(wid >= NW // 2)
            def _b():
                for s in range(n_sub):
                    pltpu.async_copy(
                        tabB_hbm.at[idx_v.at[pl.ds(s * SUB, SUB)]],
                        w_v.at[pl.ds(s * SUB, SUB)], sem)

        def drain_gathers(w_v, sem):
            pltpu.make_async_copy(idx_hbm.at[pl.ds(0, w_win)], w_v, sem).wait()

        def drain_idx(idx_v, sem):
            pltpu.make_async_copy(idx_hbm.at[pl.ds(0, w_win)], idx_v, sem).wait()

        def drain_out(g1_v, g2_v, sem):
            pltpu.make_async_copy(o1_hbm.at[pl.ds(0, w_win)], g1_v, sem).wait()
            pltpu.make_async_copy(o2_hbm.at[pl.ds(0, w_win)], g2_v, sem).wait()

        def extract(w_v, g1_v, g2_v):
            @pl.loop(0, w_win // 16)
            def _ext(q):
                word = w_v[pl.ds(q * 16, 16)]
                g1_v[pl.ds(q * 16, 16)] = lax.bitcast_convert_type(
                    word << 16, jnp.float32)
                g2_v[pl.ds(q * 16, 16)] = lax.bitcast_convert_type(
                    word & jnp.int32(-65536), jnp.float32)

        pltpu.sync_copy(idx_hbm.at[pl.ds(base, w_win)], idx0)
        fire_gathers(idx0, w0, sg0)
        pltpu.async_copy(idx_hbm.at[pl.ds(base + w_win, w_win)], idx1, si1)

        @pl.loop(0, n2)
        def _pair(i):
            a_off = base + (2 * i) * w_win
            b_off = a_off + w_win
            n_off = lax.rem(b_off + w_win, ntot)
            m_off = lax.rem(b_off + 2 * w_win, ntot)

            drain_idx(idx1, si1)
            fire_gathers(idx1, w1, sg1)

            drain_gathers(w0, sg0)
            pltpu.async_copy(idx_hbm.at[pl.ds(n_off, w_win)], idx0, si0)

            @pl.when(i > 0)
            def _dr0():
                drain_out(g1_0, g2_0, so0)

            extract(w0, g1_0, g2_0)
            pltpu.async_copy(g1_0, o1_hbm.at[pl.ds(a_off, w_win)], so0)
            pltpu.async_copy(g2_0, o2_hbm.at[pl.ds(a_off, w_win)], so0)

            drain_idx(idx0, si0)
            fire_gathers(idx0, w0, sg0)

            drain_gathers(w1, sg1)
            pltpu.async_copy(idx_hbm.at[pl.ds(m_off, w_win)], idx1, si1)

            @pl.when(i > 0)
            def _dr1():
                drain_out(g1_1, g2_1, so1)

            extract(w1, g1_1, g2_1)
            pltpu.async_copy(g1_1, o1_hbm.at[pl.ds(b_off, w_win)], so1)
            pltpu.async_copy(g2_1, o2_hbm.at[pl.ds(b_off, w_win)], so1)

        drain_idx(idx1, si1)
        drain_gathers(w0, sg0)
        drain_out(g1_0, g2_0, so0)
        drain_out(g1_1, g2_1, so1)

    return k(tabA_flat, tabB_flat, ewords)


def _loss_body(g1r, g2r, lo):
    c0 = K * (1.0 / N_DATA)
    c = c0 + EPS
    kcol = lax.broadcasted_iota(jnp.int32, (BSZ, KPAD), 1)
    valid = kcol < KP1
    tot = jnp.float32(0.0)
    for gr in (g2r, g1r):
        e = jnp.exp(gr[...] * (1.0 / T))
        e = jnp.where(valid, e, 0.0)
        z = jnp.sum(e) * (float(N_DATA) / (BSZ * KP1))
        px = e / z
        term = jnp.where(kcol == 0,
                         jnp.log(px / (px + c)),
                         jnp.log(c0 / (px + c)))
        term = jnp.where(valid, term, 0.0)
        tot = tot + (-jnp.sum(term) / BSZ)
    lo[...] = jnp.reshape(tot, (1, 1))


def _tc_loss(g1, g2):
    full = lambda s: pl.BlockSpec(s, lambda: (0,) * len(s))
    return pl.pallas_call(
        _loss_body,
        in_specs=[full((BSZ, KPAD)), full((BSZ, KPAD))],
        out_specs=full((1, 1)),
        out_shape=jax.ShapeDtypeStruct((1, 1), jnp.float32),
    )(g1, g2)


def kernel(f_s, f_t, idx, contrast_idx, W_s, b_s, W_t, b_t,
           memory_v1, memory_v2):
    idx32 = idx.astype(jnp.int32)
    cidx = contrast_idx.astype(jnp.int32)
    cpad = jnp.pad(cidx, ((0, 0), (0, KPAD - KP1)))
    half = BSZ // 2
    bmod = jnp.arange(BSZ, dtype=jnp.int32)[:, None] % half
    e_word = cpad * half + bmod
    ewords = e_word.reshape(BSZ * KPAD)

    r1, r2 = _sc_gather_rows(memory_v1, memory_v2, idx32)
    v1b, v2b, d1b, d2b = _tc_embed(
        f_s, f_t, W_s, b_s[None, :], W_t, b_t[None, :],
        idx32[:, None], idx32[None, :], r1, r2)
    new_m1, new_m2, sA, sB = _tc_big(
        memory_v1, memory_v2, v1b, v2b, d1b, d2b, idx32[None, :])
    g1f, g2f = _sc_gather_vals(sA.reshape(N_DATA * half),
                               sB.reshape(N_DATA * half), ewords)
    loss = _tc_loss(g1f.reshape(BSZ, KPAD), g2f.reshape(BSZ, KPAD))
    return loss.reshape(1), new_m1, new_m2
